# baseline (speedup 1.0000x reference)
import jax
import jax.numpy as jnp
from jax.experimental import pallas as pl
from jax.experimental.pallas import tpu as pltpu

_BM = 512


def _fused_kernel(x_ref, adj_ref, w1_ref, b1_ref, w2_ref, b2_ref,
                  out_ref, s1_ref, h2_ref):
    i = pl.program_id(0)
    nb = pl.num_programs(0) // 2

    @pl.when(i == 0)
    def _():
        s1_ref[...] = jnp.dot(x_ref[...], w1_ref[...],
                              preferred_element_type=jnp.float32)

    @pl.when(i < nb)
    def _():
        acc = jnp.dot(adj_ref[...], s1_ref[...],
                      preferred_element_type=jnp.float32)
        hb = jnp.maximum(acc + b1_ref[...], 0.0)
        h2_ref[pl.ds(i * _BM, _BM), :] = jnp.dot(
            hb, w2_ref[...], preferred_element_type=jnp.float32)

    @pl.when(i >= nb)
    def _():
        n = x_ref.shape[0]
        o = jnp.dot(adj_ref[...], h2_ref[:n, :],
                    preferred_element_type=jnp.float32)
        o = o + b2_ref[...]
        m = jnp.max(o, axis=1, keepdims=True)
        shifted = o - m
        lse = jnp.log(jnp.sum(jnp.exp(shifted), axis=1, keepdims=True))
        out_ref[...] = shifted - lse


@jax.jit
def kernel(x, adj, W1, b1, W2, b2):
    n, nfeat = x.shape
    nhid = W1.shape[1]
    nclass = W2.shape[1]
    b1r = b1.reshape(1, nhid)
    b2r = b2.reshape(1, nclass)
    nb = -(-n // _BM)

    return pl.pallas_call(
        _fused_kernel,
        grid=(2 * nb,),
        in_specs=[
            pl.BlockSpec((n, nfeat), lambda i: (0, 0)),
            pl.BlockSpec((_BM, n), lambda i: (i % nb, 0)),
            pl.BlockSpec((nfeat, nhid), lambda i: (0, 0)),
            pl.BlockSpec((1, nhid), lambda i: (0, 0)),
            pl.BlockSpec((nhid, nclass), lambda i: (0, 0)),
            pl.BlockSpec((1, nclass), lambda i: (0, 0)),
        ],
        out_specs=pl.BlockSpec(
            (_BM, nclass), lambda i: (jnp.maximum(i - nb, 0), 0)),
        out_shape=jax.ShapeDtypeStruct((n, nclass), jnp.float32),
        scratch_shapes=[
            pltpu.VMEM((n, nhid), jnp.float32),
            pltpu.VMEM((nb * _BM, nclass), jnp.float32),
        ],
        compiler_params=pltpu.CompilerParams(
            dimension_semantics=("arbitrary",)),
    )(x, adj, W1, b1r, W2, b2r)


# speedup vs baseline: 1.0019x; 1.0019x over previous
import jax
import jax.numpy as jnp
from jax.experimental import pallas as pl
from jax.experimental.pallas import tpu as pltpu

_CH = 200
_NBUF = 4


def _fused_kernel(x_ref, adj_ref, w1_ref, b1_ref, w2_ref, b2_ref,
                  out_ref, s1_ref, h2_ref, buf_ref, sem_ref):
    i = pl.program_id(0)
    nsteps = pl.num_programs(0)
    nb = nsteps // 2
    n = x_ref.shape[0]

    def chunk_copy(g):
        rows = (g % nb) * _CH
        slot = g % _NBUF
        return pltpu.make_async_copy(
            adj_ref.at[pl.ds(rows, _CH), :], buf_ref.at[slot],
            sem_ref.at[slot])

    @pl.when(i == 0)
    def _():
        for g in range(_NBUF):
            chunk_copy(g).start()
        s1_ref[...] = jnp.dot(x_ref[...], w1_ref[...],
                              preferred_element_type=jnp.float32)

    @pl.when(jnp.logical_and(i > 0, i + _NBUF - 1 < nsteps))
    def _():
        chunk_copy(i + _NBUF - 1).start()

    chunk_copy(i).wait()
    a = buf_ref[i % _NBUF]

    @pl.when(i < nb)
    def _():
        acc = jnp.dot(a, s1_ref[...], preferred_element_type=jnp.float32)
        hb = jnp.maximum(acc + b1_ref[...], 0.0)
        h2_ref[pl.ds((i % nb) * _CH, _CH), :] = jnp.dot(
            hb, w2_ref[...], preferred_element_type=jnp.float32)

    @pl.when(i >= nb)
    def _():
        o = jnp.dot(a, h2_ref[...], preferred_element_type=jnp.float32)
        o = o + b2_ref[...]
        m = jnp.max(o, axis=1, keepdims=True)
        shifted = o - m
        lse = jnp.log(jnp.sum(jnp.exp(shifted), axis=1, keepdims=True))
        out_ref[...] = shifted - lse


@jax.jit
def kernel(x, adj, W1, b1, W2, b2):
    n, nfeat = x.shape
    nhid = W1.shape[1]
    nclass = W2.shape[1]
    b1r = b1.reshape(1, nhid)
    b2r = b2.reshape(1, nclass)
    nb = n // _CH

    return pl.pallas_call(
        _fused_kernel,
        grid=(2 * nb,),
        in_specs=[
            pl.BlockSpec((n, nfeat), lambda i: (0, 0)),
            pl.BlockSpec(memory_space=pl.ANY),
            pl.BlockSpec((nfeat, nhid), lambda i: (0, 0)),
            pl.BlockSpec((1, nhid), lambda i: (0, 0)),
            pl.BlockSpec((nhid, nclass), lambda i: (0, 0)),
            pl.BlockSpec((1, nclass), lambda i: (0, 0)),
        ],
        out_specs=pl.BlockSpec(
            (_CH, nclass), lambda i: (jnp.maximum(i - nb, 0), 0)),
        out_shape=jax.ShapeDtypeStruct((n, nclass), jnp.float32),
        scratch_shapes=[
            pltpu.VMEM((n, nhid), jnp.float32),
            pltpu.VMEM((n, nclass), jnp.float32),
            pltpu.VMEM((_NBUF, _CH, n), jnp.float32),
            pltpu.SemaphoreType.DMA((_NBUF,)),
        ],
        compiler_params=pltpu.CompilerParams(
            dimension_semantics=("arbitrary",)),
    )(x, adj, W1, b1r, W2, b2r)
